# baseline (speedup 1.0000x reference)
import functools
import jax
import jax.numpy as jnp
from jax import lax
from jax.experimental import pallas as pl
from jax.experimental.pallas import tpu as pltpu
from jax.experimental.pallas import tpu_sc as plsc

F32 = jnp.float32
K = 48
NTEC = 32
NSC = 2
TPS = 16


def _make_sc_edge(nG, G, N, NP, Epad, E_real):
    CV = 128 // (16 * G)
    EP = Epad // NTEC
    nch = EP // K
    RPT = NP // TPS

    mesh = plsc.VectorSubcoreMesh(core_axis_name="c", subcore_axis_name="s")

    @functools.partial(
        pl.kernel,
        out_type=(
            jax.ShapeDtypeStruct((NSC * nG * NP, 128), F32),
            jax.ShapeDtypeStruct((NSC * nG * NP, 16), F32),
        ),
        mesh=mesh,
        compiler_params=pltpu.CompilerParams(
            needs_layout_passes=False, use_tc_tiling_on_sc=False),
        scratch_types=[
            pltpu.VMEM_SHARED((NP, 128), F32),
            pltpu.VMEM_SHARED((NP, 16), F32),
            pltpu.VMEM((K,), jnp.int32),
            pltpu.VMEM((K,), jnp.int32),
            pltpu.VMEM((K,), jnp.int32),
            pltpu.VMEM((K,), jnp.int32),
            pltpu.VMEM((K,), jnp.int32),
            pltpu.VMEM((K,), jnp.int32),
            pltpu.VMEM((K,), jnp.int32),
            pltpu.VMEM((K,), jnp.int32),
            pltpu.VMEM((K,), jnp.int32),
            pltpu.VMEM((K,), jnp.int32),
            pltpu.VMEM((K, 128), F32),
            pltpu.VMEM((K, 128), F32),
            pltpu.VMEM((K, 128), F32),
            pltpu.VMEM((K, 128), F32),
            pltpu.VMEM((K, 128), F32),
            pltpu.VMEM((K, 128), F32),
            pltpu.VMEM((K, 16), F32),
            pltpu.VMEM((K, 16), F32),
            pltpu.VMEM((16, 16), F32),
            pltpu.VMEM((128,), F32),
            pltpu.SemaphoreType.DMA,
            pltpu.SemaphoreType.DMA,
            pltpu.SemaphoreType.DMA,
            pltpu.SemaphoreType.DMA,
            pltpu.SemaphoreType.DMA,
            pltpu.SemaphoreType.DMA,
            pltpu.SemaphoreType.DMA,
            pltpu.SemaphoreType.DMA,
        ],
    )
    def edge_kernel(xl_hbm, xr_hbm, src_hbm, dst_hbm, att_hbm, zf_hbm, z16_hbm,
                    acc_out, s_out,
                    acc_sp, s_sp, srcv0, srcv1, dstv0, dstv1, idxl0, idxl1,
                    idxr0, idxr1, sdst0, sdst1, xlv0, xlv1, xrv0, xrv1,
                    msgv0, msgv1, wv0, wv1, tbuf, attc,
                    gsl0, gsl1, gsr0, gsr1, ssa0, ssa1, sss0, sss1):
        c = lax.axis_index("c")
        t = lax.axis_index("s")
        wid = c * TPS + t
        ebase = wid * EP
        rbase = t * RPT
        lane = lax.iota(jnp.int32, 16)
        Ch = 128 // G
        bufs0 = (srcv0, dstv0, idxl0, idxr0, sdst0, xlv0, xrv0, msgv0, wv0,
                 gsl0, gsr0, ssa0, sss0)
        bufs1 = (srcv1, dstv1, idxl1, idxr1, sdst1, xlv1, xrv1, msgv1, wv1,
                 gsl1, gsr1, ssa1, sss1)

        def wz_body(i, _):
            wv0[i, pl.ds(0, 16)] = jnp.zeros((16,), F32)
            wv1[i, pl.ds(0, 16)] = jnp.zeros((16,), F32)
            return 0

        lax.fori_loop(0, K, wz_body, 0)

        def load_idx_and_gather(k, gN, bufs):
            srcv, dstv, idxl, idxr, _, xlv, xrv, _, _, gsl, gsr, _, _ = bufs
            base = ebase + k * K
            pltpu.sync_copy(src_hbm.at[pl.ds(base, K)], srcv)
            pltpu.sync_copy(dst_hbm.at[pl.ds(base, K)], dstv)
            for j in range(K // 16):
                sl = pl.ds(j * 16, 16)
                idxl[sl] = srcv[sl] + gN
                idxr[sl] = dstv[sl] + gN
            pltpu.async_copy(xl_hbm.at[idxl], xlv, gsl)
            pltpu.async_copy(xr_hbm.at[idxr], xrv, gsr)

        def group_body(g, _):
            gN = g * N
            pltpu.sync_copy(att_hbm.at[g], attc)
            attvecs = [attc[pl.ds(j * 16, 16)] for j in range(8)]
            pltpu.sync_copy(zf_hbm.at[pl.ds(rbase, RPT)],
                            acc_sp.at[pl.ds(rbase, RPT)])
            pltpu.sync_copy(z16_hbm.at[pl.ds(rbase, RPT)],
                            s_sp.at[pl.ds(rbase, RPT)])
            plsc.subcore_barrier()

            def chunk_slot(k, bufs, nbufs):
                (srcv, dstv, idxl, idxr, sdst, xlv, xrv, msgv, wv,
                 gsl, gsr, ssa, sss) = bufs
                base = ebase + k * K
                pltpu.make_async_copy(xl_hbm.at[idxl], xlv, gsl).wait()
                pltpu.make_async_copy(xr_hbm.at[idxr], xrv, gsr).wait()

                @pl.when(k + 1 < nch)
                def _():
                    load_idx_and_gather(k + 1, gN, nbufs)

                def blk_body(eb, _):
                    rows = eb * 16 + lane
                    mvec = jnp.where(base + rows < E_real, 1.0, 0.0
                                     ).astype(F32)
                    for h in range(G):
                        for j in range(16):
                            e = eb * 16 + j
                            psum = None
                            for q in range(CV):
                                sl = pl.ds((h * CV + q) * 16, 16)
                                z = xlv[e, sl] + xrv[e, sl]
                                lk = jnp.maximum(z, z * 0.2)
                                term = lk * attvecs[h * CV + q]
                                psum = term if psum is None else psum + term
                            tbuf[j, pl.ds(0, 16)] = psum
                        cols = [
                            plsc.load_gather(
                                tbuf,
                                [lane, jnp.full((16,), j, jnp.int32)])
                            for j in range(16)
                        ]
                        while len(cols) > 1:
                            cols = [cols[i] + cols[i + 1]
                                    for i in range(0, len(cols), 2)]
                        w = jnp.exp(cols[0]) * mvec
                        plsc.store_scatter(
                            wv, [rows, jnp.full((16,), h, jnp.int32)], w)
                        for j in range(16):
                            e = eb * 16 + j
                            ws = w[j]
                            for q in range(CV):
                                sl = pl.ds((h * CV + q) * 16, 16)
                                msgv[e, sl] = xlv[e, sl] * ws
                    return 0

                lax.fori_loop(0, K // 16, blk_body, 0)

                pltpu.sync_copy(msgv, acc_sp.at[dstv], add=True)
                pltpu.sync_copy(wv, s_sp.at[dstv], add=True)

            load_idx_and_gather(0, gN, bufs0)

            def pair_body(i, _):
                chunk_slot(2 * i, bufs0, bufs1)
                chunk_slot(2 * i + 1, bufs1, bufs0)
                return 0

            lax.fori_loop(0, nch // 2, pair_body, 0)
            plsc.subcore_barrier()
            obase = (c * nG + g) * NP + rbase
            pltpu.sync_copy(acc_sp.at[pl.ds(rbase, RPT)],
                            acc_out.at[pl.ds(obase, RPT)])
            pltpu.sync_copy(s_sp.at[pl.ds(rbase, RPT)],
                            s_out.at[pl.ds(obase, RPT)])
            plsc.subcore_barrier()
            return 0

        lax.fori_loop(0, nG, group_body, 0)

    return edge_kernel


def _mm_in_body(x_ref, wl_ref, wr_ref, bl_ref, br_ref, xl_ref, xr_ref):
    xb = x_ref[...]
    xl_ref[0] = jnp.dot(xb, wl_ref[...], preferred_element_type=F32) + bl_ref[0]
    xr_ref[0] = jnp.dot(xb, wr_ref[...], preferred_element_type=F32) + br_ref[0]


def _mm_in(x, Wl, Wr, bl, br, nG, R):
    N = x.shape[0]
    HIDD = x.shape[1]
    grid = (nG, N // R)
    out_shape = [
        jax.ShapeDtypeStruct((nG, N, 128), F32),
        jax.ShapeDtypeStruct((nG, N, 128), F32),
    ]
    return pl.pallas_call(
        _mm_in_body,
        grid=grid,
        in_specs=[
            pl.BlockSpec((R, HIDD), lambda g, r: (r, 0)),
            pl.BlockSpec((HIDD, 128), lambda g, r: (0, g)),
            pl.BlockSpec((HIDD, 128), lambda g, r: (0, g)),
            pl.BlockSpec((1, 1, 128), lambda g, r: (g, 0, 0)),
            pl.BlockSpec((1, 1, 128), lambda g, r: (g, 0, 0)),
        ],
        out_specs=[
            pl.BlockSpec((1, R, 128), lambda g, r: (g, r, 0)),
            pl.BlockSpec((1, R, 128), lambda g, r: (g, r, 0)),
        ],
        out_shape=out_shape,
    )(x, Wl, Wr, bl.reshape(nG, 1, 128), br.reshape(nG, 1, 128))


def _elu(v):
    return jnp.where(v > 0, v, jnp.exp(v) - 1.0)


def _norm_block(acc, s, bias_ref, nG, G, R):
    C = 128 // G
    col = lax.broadcasted_iota(jnp.int32, (R, 128), 1)
    outs = []
    for g in range(nG):
        den = jnp.zeros((R, 128), F32)
        for h in range(G):
            sh = s[g, :, h:h + 1]
            m = jnp.logical_and(col >= h * C, col < (h + 1) * C)
            den = jnp.where(m, sh, den)
        hg = acc[g] / den + bias_ref[g]
        outs.append(_elu(hg))
    return jnp.concatenate(outs, axis=1)


def _norm1_mm2_body(a0_ref, a1_ref, s0_ref, s1_ref, b1_ref, w2_ref, b2_ref,
                    h1_ref, xl2_ref, xr2_ref, *, R):
    acc = a0_ref[...] + a1_ref[...]
    s = s0_ref[...] + s1_ref[...]
    h1 = _norm_block(acc, s, b1_ref[...], 4, 2, R)
    h1_ref[...] = h1
    xw = jnp.dot(h1, w2_ref[...], preferred_element_type=F32)
    b2 = b2_ref[...]
    for q in range(2):
        xl2_ref[q] = xw[:, q * 128:(q + 1) * 128] + b2[q]
        xr2_ref[q] = xw[:, 256 + q * 128:256 + (q + 1) * 128] + b2[2 + q]


def _norm1_mm2(acc1, s1, bias1, Wl2, Wr2, bl2, br2, N, R):
    W2 = jnp.concatenate([Wl2, Wr2], axis=1)
    b2 = jnp.concatenate([bl2, br2]).reshape(4, 128)
    grid = (N // R,)
    out_shape = [
        jax.ShapeDtypeStruct((N, 512), F32),
        jax.ShapeDtypeStruct((2, N, 128), F32),
        jax.ShapeDtypeStruct((2, N, 128), F32),
    ]
    return pl.pallas_call(
        functools.partial(_norm1_mm2_body, R=R),
        grid=grid,
        in_specs=[
            pl.BlockSpec((4, R, 128), lambda r: (0, r, 0)),
            pl.BlockSpec((4, R, 128), lambda r: (0, r, 0)),
            pl.BlockSpec((4, R, 16), lambda r: (0, r, 0)),
            pl.BlockSpec((4, R, 16), lambda r: (0, r, 0)),
            pl.BlockSpec((4, 128), lambda r: (0, 0)),
            pl.BlockSpec((512, 512), lambda r: (0, 0)),
            pl.BlockSpec((4, 128), lambda r: (0, 0)),
        ],
        out_specs=[
            pl.BlockSpec((R, 512), lambda r: (r, 0)),
            pl.BlockSpec((2, R, 128), lambda r: (0, r, 0)),
            pl.BlockSpec((2, R, 128), lambda r: (0, r, 0)),
        ],
        out_shape=out_shape,
    )(acc1[0], acc1[1], s1[0], s1[1], bias1.reshape(4, 128), W2, b2)


def _norm2_jk_body(x_ref, h1_ref, a0_ref, a1_ref, s0_ref, s1_ref, b2_ref,
                   wjk_ref, bjk_ref, out_ref, *, R):
    acc = a0_ref[...] + a1_ref[...]
    s = s0_ref[...] + s1_ref[...]
    h2 = _norm_block(acc, s, b2_ref[...], 2, 4, R)
    w = wjk_ref[...]
    o = jnp.dot(x_ref[...], w[0:128], preferred_element_type=F32)
    o = o + jnp.dot(h1_ref[...], w[128:640], preferred_element_type=F32)
    o = o + jnp.dot(h2, w[640:896], preferred_element_type=F32)
    out_ref[...] = o + bjk_ref[...]


def _norm2_jk(x, h1, acc2a, acc2b, s2a, s2b, bias2, Wjk, bjk, N, R):
    grid = (N // R,)
    return pl.pallas_call(
        functools.partial(_norm2_jk_body, R=R),
        grid=grid,
        in_specs=[
            pl.BlockSpec((R, 128), lambda r: (r, 0)),
            pl.BlockSpec((R, 512), lambda r: (r, 0)),
            pl.BlockSpec((2, R, 128), lambda r: (0, r, 0)),
            pl.BlockSpec((2, R, 128), lambda r: (0, r, 0)),
            pl.BlockSpec((2, R, 16), lambda r: (0, r, 0)),
            pl.BlockSpec((2, R, 16), lambda r: (0, r, 0)),
            pl.BlockSpec((2, 128), lambda r: (0, 0)),
            pl.BlockSpec((896, 128), lambda r: (0, 0)),
            pl.BlockSpec((1, 128), lambda r: (0, 0)),
        ],
        out_specs=pl.BlockSpec((R, 128), lambda r: (r, 0)),
        out_shape=jax.ShapeDtypeStruct((N, 128), F32),
    )(x, h1, acc2a, acc2b, s2a, s2b, bias2.reshape(2, 128), Wjk,
      bjk.reshape(1, 128))


def kernel(x, edge_index, Wl1, bl1, Wr1, br1, att1, bias1,
           Wl2, bl2, Wr2, br2, att2, bias2, Wjk, bjk):
    N = x.shape[0]
    E = edge_index.shape[1]
    E_real = E + N
    q = NTEC * K * 2
    Epad = ((E_real + q - 1) // q) * q
    NP = ((N + TPS * 8 - 1) // (TPS * 8)) * (TPS * 8)
    R = 1000

    loop = jnp.arange(N, dtype=edge_index.dtype)
    pad = jnp.zeros((Epad - E_real,), dtype=edge_index.dtype)
    src = jnp.concatenate([edge_index[0], loop, pad])
    dst = jnp.concatenate([edge_index[1], loop, pad])

    zf = jnp.zeros((NP, 128), F32)
    z16 = jnp.zeros((NP, 16), F32)

    xl1, xr1 = _mm_in(x, Wl1, Wr1, bl1, br1, 4, R)
    att1g = att1.reshape(4, 128)
    ek1 = _make_sc_edge(4, 2, N, NP, Epad, E_real)
    acc1, s1 = ek1(xl1.reshape(4 * N, 128), xr1.reshape(4 * N, 128),
                   src, dst, att1g, zf, z16)
    acc1 = acc1.reshape(NSC, 4, NP, 128)[:, :, :N]
    s1 = s1.reshape(NSC, 4, NP, 16)[:, :, :N]

    h1, xl2, xr2 = _norm1_mm2((acc1[0], acc1[1]), (s1[0], s1[1]), bias1,
                              Wl2, Wr2, bl2, br2, N, R)

    att2g = att2.reshape(2, 128)
    ek2 = _make_sc_edge(2, 4, N, NP, Epad, E_real)
    acc2, s2 = ek2(xl2.reshape(2 * N, 128), xr2.reshape(2 * N, 128),
                   src, dst, att2g, zf, z16)
    acc2 = acc2.reshape(NSC, 2, NP, 128)[:, :, :N]
    s2 = s2.reshape(NSC, 2, NP, 16)[:, :, :N]

    return _norm2_jk(x, h1, acc2[0], acc2[1], s2[0], s2[1], bias2,
                     Wjk, bjk, N, R)


# speedup vs baseline: 1.0869x; 1.0869x over previous
import functools
import jax
import jax.numpy as jnp
from jax import lax
from jax.experimental import pallas as pl
from jax.experimental.pallas import tpu as pltpu
from jax.experimental.pallas import tpu_sc as plsc

F32 = jnp.float32
K = 64
NTEC = 32
NSC = 2
TPS = 16


def _make_sc_edge(nG, G, N, NP, Epad, E_real):
    CV = 128 // (16 * G)
    EP = Epad // NTEC
    nch = EP // K
    RPT = NP // TPS

    mesh = plsc.VectorSubcoreMesh(core_axis_name="c", subcore_axis_name="s")

    @functools.partial(
        pl.kernel,
        out_type=(
            jax.ShapeDtypeStruct((NSC * nG * NP, 128), F32),
            jax.ShapeDtypeStruct((NSC * nG * NP, 16), F32),
        ),
        mesh=mesh,
        compiler_params=pltpu.CompilerParams(
            needs_layout_passes=False, use_tc_tiling_on_sc=False),
        scratch_types=[
            pltpu.VMEM_SHARED((NP, 128), F32),
            pltpu.VMEM_SHARED((NP, 16), F32),
            pltpu.VMEM((K,), jnp.int32),
            pltpu.VMEM((K,), jnp.int32),
            pltpu.VMEM((K,), jnp.int32),
            pltpu.VMEM((K,), jnp.int32),
            pltpu.VMEM((K,), jnp.int32),
            pltpu.VMEM((K,), jnp.int32),
            pltpu.VMEM((K,), jnp.int32),
            pltpu.VMEM((K,), jnp.int32),
            pltpu.VMEM((K, 128), F32),
            pltpu.VMEM((K, 128), F32),
            pltpu.VMEM((K, 128), F32),
            pltpu.VMEM((K, 128), F32),
            pltpu.VMEM((K, 16), F32),
            pltpu.VMEM((K, 16), F32),
            pltpu.VMEM((16, 16), F32),
            pltpu.VMEM((128,), F32),
            pltpu.SemaphoreType.DMA,
            pltpu.SemaphoreType.DMA,
            pltpu.SemaphoreType.DMA,
            pltpu.SemaphoreType.DMA,
        ],
    )
    def edge_kernel(xl_hbm, xr_hbm, src_hbm, dst_hbm, att_hbm, zf_hbm, z16_hbm,
                    acc_out, s_out,
                    acc_sp, s_sp, srcv0, srcv1, dstv0, dstv1, idxl0, idxl1,
                    idxr0, idxr1, xlv0, xlv1, xrv0, xrv1, wv0, wv1, tbuf,
                    attc, gsl0, gsl1, gsr0, gsr1):
        c = lax.axis_index("c")
        t = lax.axis_index("s")
        wid = c * TPS + t
        ebase = wid * EP
        rbase = t * RPT
        lane = lax.iota(jnp.int32, 16)
        Ch = 128 // G
        bufs0 = (srcv0, dstv0, idxl0, idxr0, xlv0, xrv0, wv0, gsl0, gsr0)
        bufs1 = (srcv1, dstv1, idxl1, idxr1, xlv1, xrv1, wv1, gsl1, gsr1)

        def wz_body(i, _):
            wv0[i, pl.ds(0, 16)] = jnp.zeros((16,), F32)
            wv1[i, pl.ds(0, 16)] = jnp.zeros((16,), F32)
            return 0

        lax.fori_loop(0, K, wz_body, 0)

        def load_idx_and_gather(k, gN, bufs):
            srcv, dstv, idxl, idxr, xlv, xrv, _, gsl, gsr = bufs
            base = ebase + k * K
            pltpu.sync_copy(src_hbm.at[pl.ds(base, K)], srcv)
            pltpu.sync_copy(dst_hbm.at[pl.ds(base, K)], dstv)
            for j in range(K // 16):
                sl = pl.ds(j * 16, 16)
                idxl[sl] = srcv[sl] + gN
                idxr[sl] = dstv[sl] + gN
            pltpu.async_copy(xl_hbm.at[idxl], xlv, gsl)
            pltpu.async_copy(xr_hbm.at[idxr], xrv, gsr)

        def group_body(g, _):
            gN = g * N
            pltpu.sync_copy(att_hbm.at[g], attc)
            attvecs = [attc[pl.ds(j * 16, 16)] for j in range(8)]
            pltpu.sync_copy(zf_hbm.at[pl.ds(rbase, RPT)],
                            acc_sp.at[pl.ds(rbase, RPT)])
            pltpu.sync_copy(z16_hbm.at[pl.ds(rbase, RPT)],
                            s_sp.at[pl.ds(rbase, RPT)])
            plsc.subcore_barrier()

            def chunk_slot(k, bufs, nbufs):
                srcv, dstv, idxl, idxr, xlv, xrv, wv, gsl, gsr = bufs
                base = ebase + k * K
                pltpu.make_async_copy(xl_hbm.at[idxl], xlv, gsl).wait()
                pltpu.make_async_copy(xr_hbm.at[idxr], xrv, gsr).wait()

                @pl.when(k + 1 < nch)
                def _():
                    load_idx_and_gather(k + 1, gN, nbufs)

                def blk_body(eb, _):
                    rows = eb * 16 + lane
                    mvec = jnp.where(base + rows < E_real, 1.0, 0.0
                                     ).astype(F32)
                    for h in range(G):
                        for j in range(16):
                            e = eb * 16 + j
                            psum = None
                            for q in range(CV):
                                sl = pl.ds((h * CV + q) * 16, 16)
                                z = xlv[e, sl] + xrv[e, sl]
                                lk = jnp.maximum(z, z * 0.2)
                                term = lk * attvecs[h * CV + q]
                                psum = term if psum is None else psum + term
                            tbuf[j, pl.ds(0, 16)] = psum
                        cols = [
                            plsc.load_gather(
                                tbuf,
                                [lane, jnp.full((16,), j, jnp.int32)])
                            for j in range(16)
                        ]
                        while len(cols) > 1:
                            cols = [cols[i] + cols[i + 1]
                                    for i in range(0, len(cols), 2)]
                        w = jnp.exp(cols[0]) * mvec
                        plsc.store_scatter(
                            wv, [rows, jnp.full((16,), h, jnp.int32)], w)
                        for j in range(16):
                            e = eb * 16 + j
                            ws = w[j]
                            for q in range(CV):
                                sl = pl.ds((h * CV + q) * 16, 16)
                                xrv[e, sl] = xlv[e, sl] * ws
                    return 0

                lax.fori_loop(0, K // 16, blk_body, 0)

                pltpu.sync_copy(xrv, acc_sp.at[dstv], add=True)
                pltpu.sync_copy(wv, s_sp.at[dstv], add=True)

            load_idx_and_gather(0, gN, bufs0)

            def pair_body(i, _):
                chunk_slot(2 * i, bufs0, bufs1)
                chunk_slot(2 * i + 1, bufs1, bufs0)
                return 0

            lax.fori_loop(0, nch // 2, pair_body, 0)
            plsc.subcore_barrier()
            obase = (c * nG + g) * NP + rbase
            pltpu.sync_copy(acc_sp.at[pl.ds(rbase, RPT)],
                            acc_out.at[pl.ds(obase, RPT)])
            pltpu.sync_copy(s_sp.at[pl.ds(rbase, RPT)],
                            s_out.at[pl.ds(obase, RPT)])
            plsc.subcore_barrier()
            return 0

        lax.fori_loop(0, nG, group_body, 0)

    return edge_kernel


def _mm_in_body(x_ref, wl_ref, wr_ref, bl_ref, br_ref, xl_ref, xr_ref):
    xb = x_ref[...]
    xl_ref[0] = jnp.dot(xb, wl_ref[...], preferred_element_type=F32) + bl_ref[0]
    xr_ref[0] = jnp.dot(xb, wr_ref[...], preferred_element_type=F32) + br_ref[0]


def _mm_in(x, Wl, Wr, bl, br, nG, R):
    N = x.shape[0]
    HIDD = x.shape[1]
    grid = (nG, N // R)
    out_shape = [
        jax.ShapeDtypeStruct((nG, N, 128), F32),
        jax.ShapeDtypeStruct((nG, N, 128), F32),
    ]
    return pl.pallas_call(
        _mm_in_body,
        grid=grid,
        in_specs=[
            pl.BlockSpec((R, HIDD), lambda g, r: (r, 0)),
            pl.BlockSpec((HIDD, 128), lambda g, r: (0, g)),
            pl.BlockSpec((HIDD, 128), lambda g, r: (0, g)),
            pl.BlockSpec((1, 1, 128), lambda g, r: (g, 0, 0)),
            pl.BlockSpec((1, 1, 128), lambda g, r: (g, 0, 0)),
        ],
        out_specs=[
            pl.BlockSpec((1, R, 128), lambda g, r: (g, r, 0)),
            pl.BlockSpec((1, R, 128), lambda g, r: (g, r, 0)),
        ],
        out_shape=out_shape,
    )(x, Wl, Wr, bl.reshape(nG, 1, 128), br.reshape(nG, 1, 128))


def _elu(v):
    return jnp.where(v > 0, v, jnp.exp(v) - 1.0)


def _norm_block(acc, s, bias_ref, nG, G, R):
    C = 128 // G
    col = lax.broadcasted_iota(jnp.int32, (R, 128), 1)
    outs = []
    for g in range(nG):
        den = jnp.zeros((R, 128), F32)
        for h in range(G):
            sh = s[g, :, h:h + 1]
            m = jnp.logical_and(col >= h * C, col < (h + 1) * C)
            den = jnp.where(m, sh, den)
        hg = acc[g] / den + bias_ref[g]
        outs.append(_elu(hg))
    return jnp.concatenate(outs, axis=1)


def _norm1_mm2_body(a0_ref, a1_ref, s0_ref, s1_ref, b1_ref, w2_ref, b2_ref,
                    h1_ref, xl2_ref, xr2_ref, *, R):
    acc = a0_ref[...] + a1_ref[...]
    s = s0_ref[...] + s1_ref[...]
    h1 = _norm_block(acc, s, b1_ref[...], 4, 2, R)
    h1_ref[...] = h1
    xw = jnp.dot(h1, w2_ref[...], preferred_element_type=F32)
    b2 = b2_ref[...]
    for q in range(2):
        xl2_ref[q] = xw[:, q * 128:(q + 1) * 128] + b2[q]
        xr2_ref[q] = xw[:, 256 + q * 128:256 + (q + 1) * 128] + b2[2 + q]


def _norm1_mm2(acc1, s1, bias1, Wl2, Wr2, bl2, br2, N, R):
    W2 = jnp.concatenate([Wl2, Wr2], axis=1)
    b2 = jnp.concatenate([bl2, br2]).reshape(4, 128)
    grid = (N // R,)
    out_shape = [
        jax.ShapeDtypeStruct((N, 512), F32),
        jax.ShapeDtypeStruct((2, N, 128), F32),
        jax.ShapeDtypeStruct((2, N, 128), F32),
    ]
    return pl.pallas_call(
        functools.partial(_norm1_mm2_body, R=R),
        grid=grid,
        in_specs=[
            pl.BlockSpec((4, R, 128), lambda r: (0, r, 0)),
            pl.BlockSpec((4, R, 128), lambda r: (0, r, 0)),
            pl.BlockSpec((4, R, 16), lambda r: (0, r, 0)),
            pl.BlockSpec((4, R, 16), lambda r: (0, r, 0)),
            pl.BlockSpec((4, 128), lambda r: (0, 0)),
            pl.BlockSpec((512, 512), lambda r: (0, 0)),
            pl.BlockSpec((4, 128), lambda r: (0, 0)),
        ],
        out_specs=[
            pl.BlockSpec((R, 512), lambda r: (r, 0)),
            pl.BlockSpec((2, R, 128), lambda r: (0, r, 0)),
            pl.BlockSpec((2, R, 128), lambda r: (0, r, 0)),
        ],
        out_shape=out_shape,
    )(acc1[0], acc1[1], s1[0], s1[1], bias1.reshape(4, 128), W2, b2)


def _norm2_jk_body(x_ref, h1_ref, a0_ref, a1_ref, s0_ref, s1_ref, b2_ref,
                   wjk_ref, bjk_ref, out_ref, *, R):
    acc = a0_ref[...] + a1_ref[...]
    s = s0_ref[...] + s1_ref[...]
    h2 = _norm_block(acc, s, b2_ref[...], 2, 4, R)
    w = wjk_ref[...]
    o = jnp.dot(x_ref[...], w[0:128], preferred_element_type=F32)
    o = o + jnp.dot(h1_ref[...], w[128:640], preferred_element_type=F32)
    o = o + jnp.dot(h2, w[640:896], preferred_element_type=F32)
    out_ref[...] = o + bjk_ref[...]


def _norm2_jk(x, h1, acc2a, acc2b, s2a, s2b, bias2, Wjk, bjk, N, R):
    grid = (N // R,)
    return pl.pallas_call(
        functools.partial(_norm2_jk_body, R=R),
        grid=grid,
        in_specs=[
            pl.BlockSpec((R, 128), lambda r: (r, 0)),
            pl.BlockSpec((R, 512), lambda r: (r, 0)),
            pl.BlockSpec((2, R, 128), lambda r: (0, r, 0)),
            pl.BlockSpec((2, R, 128), lambda r: (0, r, 0)),
            pl.BlockSpec((2, R, 16), lambda r: (0, r, 0)),
            pl.BlockSpec((2, R, 16), lambda r: (0, r, 0)),
            pl.BlockSpec((2, 128), lambda r: (0, 0)),
            pl.BlockSpec((896, 128), lambda r: (0, 0)),
            pl.BlockSpec((1, 128), lambda r: (0, 0)),
        ],
        out_specs=pl.BlockSpec((R, 128), lambda r: (r, 0)),
        out_shape=jax.ShapeDtypeStruct((N, 128), F32),
    )(x, h1, acc2a, acc2b, s2a, s2b, bias2.reshape(2, 128), Wjk,
      bjk.reshape(1, 128))


def kernel(x, edge_index, Wl1, bl1, Wr1, br1, att1, bias1,
           Wl2, bl2, Wr2, br2, att2, bias2, Wjk, bjk):
    N = x.shape[0]
    E = edge_index.shape[1]
    E_real = E + N
    q = NTEC * K * 2
    Epad = ((E_real + q - 1) // q) * q
    NP = ((N + TPS * 8 - 1) // (TPS * 8)) * (TPS * 8)
    R = 1000

    loop = jnp.arange(N, dtype=edge_index.dtype)
    pad = jnp.zeros((Epad - E_real,), dtype=edge_index.dtype)
    src = jnp.concatenate([edge_index[0], loop, pad])
    dst = jnp.concatenate([edge_index[1], loop, pad])

    zf = jnp.zeros((NP, 128), F32)
    z16 = jnp.zeros((NP, 16), F32)

    xl1, xr1 = _mm_in(x, Wl1, Wr1, bl1, br1, 4, R)
    att1g = att1.reshape(4, 128)
    ek1 = _make_sc_edge(4, 2, N, NP, Epad, E_real)
    acc1, s1 = ek1(xl1.reshape(4 * N, 128), xr1.reshape(4 * N, 128),
                   src, dst, att1g, zf, z16)
    acc1 = acc1.reshape(NSC, 4, NP, 128)[:, :, :N]
    s1 = s1.reshape(NSC, 4, NP, 16)[:, :, :N]

    h1, xl2, xr2 = _norm1_mm2((acc1[0], acc1[1]), (s1[0], s1[1]), bias1,
                              Wl2, Wr2, bl2, br2, N, R)

    att2g = att2.reshape(2, 128)
    ek2 = _make_sc_edge(2, 4, N, NP, Epad, E_real)
    acc2, s2 = ek2(xl2.reshape(2 * N, 128), xr2.reshape(2 * N, 128),
                   src, dst, att2g, zf, z16)
    acc2 = acc2.reshape(NSC, 2, NP, 128)[:, :, :N]
    s2 = s2.reshape(NSC, 2, NP, 16)[:, :, :N]

    return _norm2_jk(x, h1, acc2[0], acc2[1], s2[0], s2[1], bias2,
                     Wjk, bjk, N, R)


# speedup vs baseline: 1.1906x; 1.0954x over previous
import functools
import jax
import jax.numpy as jnp
from jax import lax
from jax.experimental import pallas as pl
from jax.experimental.pallas import tpu as pltpu
from jax.experimental.pallas import tpu_sc as plsc

F32 = jnp.float32
K = 64
NTEC = 32
NSC = 2
TPS = 16


def _make_sc_edge(nG, G, N, NP, Epad, E_real):
    CV = 128 // (16 * G)
    EP = Epad // NTEC
    nch = EP // K
    RPT = NP // TPS

    mesh = plsc.VectorSubcoreMesh(core_axis_name="c", subcore_axis_name="s")

    @functools.partial(
        pl.kernel,
        out_type=(
            jax.ShapeDtypeStruct((NSC * nG * NP, 128), F32),
            jax.ShapeDtypeStruct((NSC * nG * NP, 16), F32),
        ),
        mesh=mesh,
        compiler_params=pltpu.CompilerParams(
            needs_layout_passes=False, use_tc_tiling_on_sc=False),
        scratch_types=[
            pltpu.VMEM_SHARED((NP, 128), F32),
            pltpu.VMEM_SHARED((NP, 16), F32),
            pltpu.VMEM((K,), jnp.int32),
            pltpu.VMEM((K,), jnp.int32),
            pltpu.VMEM((K,), jnp.int32),
            pltpu.VMEM((K,), jnp.int32),
            pltpu.VMEM((K,), jnp.int32),
            pltpu.VMEM((K,), jnp.int32),
            pltpu.VMEM((K,), jnp.int32),
            pltpu.VMEM((K,), jnp.int32),
            pltpu.VMEM((K, 128), F32),
            pltpu.VMEM((K, 128), F32),
            pltpu.VMEM((K, 128), F32),
            pltpu.VMEM((K, 128), F32),
            pltpu.VMEM((K, 16), F32),
            pltpu.VMEM((K, 16), F32),
            pltpu.VMEM((16, 16), F32),
            pltpu.VMEM((128,), F32),
            pltpu.SemaphoreType.DMA,
            pltpu.SemaphoreType.DMA,
            pltpu.SemaphoreType.DMA,
            pltpu.SemaphoreType.DMA,
            pltpu.SemaphoreType.DMA,
            pltpu.SemaphoreType.DMA,
        ],
    )
    def edge_kernel(xl_hbm, xr_hbm, src_hbm, dst_hbm, att_hbm, zf_hbm, z16_hbm,
                    acc_out, s_out,
                    acc_sp, s_sp, srcv0, srcv1, dstv0, dstv1, idxl0, idxl1,
                    idxr0, idxr1, xlv0, xlv1, xrv0, xrv1, wv0, wv1, tbuf,
                    attc, gsl0, gsl1, gsr0, gsr1, isem0, isem1):
        c = lax.axis_index("c")
        t = lax.axis_index("s")
        wid = c * TPS + t
        ebase = wid * EP
        rbase = t * RPT
        lane = lax.iota(jnp.int32, 16)
        Ch = 128 // G
        bufs0 = (srcv0, dstv0, idxl0, idxr0, xlv0, xrv0, wv0, gsl0, gsr0,
                 isem0)
        bufs1 = (srcv1, dstv1, idxl1, idxr1, xlv1, xrv1, wv1, gsl1, gsr1,
                 isem1)

        def wz_body(i, _):
            wv0[i, pl.ds(0, 16)] = jnp.zeros((16,), F32)
            wv1[i, pl.ds(0, 16)] = jnp.zeros((16,), F32)
            return 0

        lax.fori_loop(0, K, wz_body, 0)

        def idx_load_async(k, bufs):
            srcv, dstv, isem = bufs[0], bufs[1], bufs[9]
            base = ebase + k * K
            pltpu.async_copy(src_hbm.at[pl.ds(base, K)], srcv, isem)
            pltpu.async_copy(dst_hbm.at[pl.ds(base, K)], dstv, isem)

        def build_and_gather(k, gN, bufs):
            srcv, dstv, idxl, idxr, xlv, xrv, _, gsl, gsr, isem = bufs
            base = ebase + k * K
            pltpu.make_async_copy(
                src_hbm.at[pl.ds(base, K)], srcv, isem).wait()
            pltpu.make_async_copy(
                dst_hbm.at[pl.ds(base, K)], dstv, isem).wait()
            for j in range(K // 16):
                sl = pl.ds(j * 16, 16)
                idxl[sl] = srcv[sl] + gN
                idxr[sl] = dstv[sl] + gN
            pltpu.async_copy(xl_hbm.at[idxl], xlv, gsl)
            pltpu.async_copy(xr_hbm.at[idxr], xrv, gsr)

        def group_body(g, _):
            gN = g * N
            pltpu.sync_copy(att_hbm.at[g], attc)
            attvecs = [attc[pl.ds(j * 16, 16)] for j in range(8)]
            pltpu.sync_copy(zf_hbm.at[pl.ds(rbase, RPT)],
                            acc_sp.at[pl.ds(rbase, RPT)])
            pltpu.sync_copy(z16_hbm.at[pl.ds(rbase, RPT)],
                            s_sp.at[pl.ds(rbase, RPT)])
            plsc.subcore_barrier()

            def chunk_slot(k, bufs, nbufs):
                srcv, dstv, idxl, idxr, xlv, xrv, wv, gsl, gsr, isem = bufs
                base = ebase + k * K
                pltpu.make_async_copy(xl_hbm.at[idxl], xlv, gsl).wait()
                pltpu.make_async_copy(xr_hbm.at[idxr], xrv, gsr).wait()

                @pl.when(k + 1 < nch)
                def _():
                    build_and_gather(k + 1, gN, nbufs)

                def blk_body(eb, _):
                    rows = eb * 16 + lane
                    mvec = jnp.where(base + rows < E_real, 1.0, 0.0
                                     ).astype(F32)
                    for h in range(G):
                        for j in range(16):
                            e = eb * 16 + j
                            psum = None
                            for q in range(CV):
                                sl = pl.ds((h * CV + q) * 16, 16)
                                z = xlv[e, sl] + xrv[e, sl]
                                lk = jnp.maximum(z, z * 0.2)
                                term = lk * attvecs[h * CV + q]
                                psum = term if psum is None else psum + term
                            tbuf[j, pl.ds(0, 16)] = psum
                        cols = [
                            plsc.load_gather(
                                tbuf,
                                [lane, jnp.full((16,), j, jnp.int32)])
                            for j in range(16)
                        ]
                        while len(cols) > 1:
                            cols = [cols[i] + cols[i + 1]
                                    for i in range(0, len(cols), 2)]
                        w = jnp.exp(cols[0]) * mvec
                        plsc.store_scatter(
                            wv, [rows, jnp.full((16,), h, jnp.int32)], w)
                        for j in range(16):
                            e = eb * 16 + j
                            ws = w[j]
                            for q in range(CV):
                                sl = pl.ds((h * CV + q) * 16, 16)
                                xrv[e, sl] = xlv[e, sl] * ws
                    return 0

                lax.fori_loop(0, K // 16, blk_body, 0)

                pltpu.sync_copy(xrv, acc_sp.at[dstv], add=True)
                pltpu.sync_copy(wv, s_sp.at[dstv], add=True)

                @pl.when(k + 2 < nch)
                def _():
                    idx_load_async(k + 2, bufs)

            idx_load_async(0, bufs0)
            idx_load_async(1, bufs1)
            build_and_gather(0, gN, bufs0)

            def pair_body(i, _):
                chunk_slot(2 * i, bufs0, bufs1)
                chunk_slot(2 * i + 1, bufs1, bufs0)
                return 0

            lax.fori_loop(0, nch // 2, pair_body, 0)
            plsc.subcore_barrier()
            obase = (c * nG + g) * NP + rbase
            pltpu.sync_copy(acc_sp.at[pl.ds(rbase, RPT)],
                            acc_out.at[pl.ds(obase, RPT)])
            pltpu.sync_copy(s_sp.at[pl.ds(rbase, RPT)],
                            s_out.at[pl.ds(obase, RPT)])
            plsc.subcore_barrier()
            return 0

        lax.fori_loop(0, nG, group_body, 0)

    return edge_kernel


def _mm_in_body(x_ref, wl_ref, wr_ref, bl_ref, br_ref, xl_ref, xr_ref):
    xb = x_ref[...]
    xl_ref[0] = jnp.dot(xb, wl_ref[...], preferred_element_type=F32) + bl_ref[0]
    xr_ref[0] = jnp.dot(xb, wr_ref[...], preferred_element_type=F32) + br_ref[0]


def _mm_in(x, Wl, Wr, bl, br, nG, R):
    N = x.shape[0]
    HIDD = x.shape[1]
    grid = (nG, N // R)
    out_shape = [
        jax.ShapeDtypeStruct((nG, N, 128), F32),
        jax.ShapeDtypeStruct((nG, N, 128), F32),
    ]
    return pl.pallas_call(
        _mm_in_body,
        grid=grid,
        in_specs=[
            pl.BlockSpec((R, HIDD), lambda g, r: (r, 0)),
            pl.BlockSpec((HIDD, 128), lambda g, r: (0, g)),
            pl.BlockSpec((HIDD, 128), lambda g, r: (0, g)),
            pl.BlockSpec((1, 1, 128), lambda g, r: (g, 0, 0)),
            pl.BlockSpec((1, 1, 128), lambda g, r: (g, 0, 0)),
        ],
        out_specs=[
            pl.BlockSpec((1, R, 128), lambda g, r: (g, r, 0)),
            pl.BlockSpec((1, R, 128), lambda g, r: (g, r, 0)),
        ],
        out_shape=out_shape,
    )(x, Wl, Wr, bl.reshape(nG, 1, 128), br.reshape(nG, 1, 128))


def _elu(v):
    return jnp.where(v > 0, v, jnp.exp(v) - 1.0)


def _norm_block(acc, s, bias_ref, nG, G, R):
    C = 128 // G
    col = lax.broadcasted_iota(jnp.int32, (R, 128), 1)
    outs = []
    for g in range(nG):
        den = jnp.zeros((R, 128), F32)
        for h in range(G):
            sh = s[g, :, h:h + 1]
            m = jnp.logical_and(col >= h * C, col < (h + 1) * C)
            den = jnp.where(m, sh, den)
        hg = acc[g] / den + bias_ref[g]
        outs.append(_elu(hg))
    return jnp.concatenate(outs, axis=1)


def _norm1_mm2_body(a0_ref, a1_ref, s0_ref, s1_ref, b1_ref, w2_ref, b2_ref,
                    h1_ref, xl2_ref, xr2_ref, *, R):
    acc = a0_ref[...] + a1_ref[...]
    s = s0_ref[...] + s1_ref[...]
    h1 = _norm_block(acc, s, b1_ref[...], 4, 2, R)
    h1_ref[...] = h1
    xw = jnp.dot(h1, w2_ref[...], preferred_element_type=F32)
    b2 = b2_ref[...]
    for q in range(2):
        xl2_ref[q] = xw[:, q * 128:(q + 1) * 128] + b2[q]
        xr2_ref[q] = xw[:, 256 + q * 128:256 + (q + 1) * 128] + b2[2 + q]


def _norm1_mm2(acc1, s1, bias1, Wl2, Wr2, bl2, br2, N, R):
    W2 = jnp.concatenate([Wl2, Wr2], axis=1)
    b2 = jnp.concatenate([bl2, br2]).reshape(4, 128)
    grid = (N // R,)
    out_shape = [
        jax.ShapeDtypeStruct((N, 512), F32),
        jax.ShapeDtypeStruct((2, N, 128), F32),
        jax.ShapeDtypeStruct((2, N, 128), F32),
    ]
    return pl.pallas_call(
        functools.partial(_norm1_mm2_body, R=R),
        grid=grid,
        in_specs=[
            pl.BlockSpec((4, R, 128), lambda r: (0, r, 0)),
            pl.BlockSpec((4, R, 128), lambda r: (0, r, 0)),
            pl.BlockSpec((4, R, 16), lambda r: (0, r, 0)),
            pl.BlockSpec((4, R, 16), lambda r: (0, r, 0)),
            pl.BlockSpec((4, 128), lambda r: (0, 0)),
            pl.BlockSpec((512, 512), lambda r: (0, 0)),
            pl.BlockSpec((4, 128), lambda r: (0, 0)),
        ],
        out_specs=[
            pl.BlockSpec((R, 512), lambda r: (r, 0)),
            pl.BlockSpec((2, R, 128), lambda r: (0, r, 0)),
            pl.BlockSpec((2, R, 128), lambda r: (0, r, 0)),
        ],
        out_shape=out_shape,
    )(acc1[0], acc1[1], s1[0], s1[1], bias1.reshape(4, 128), W2, b2)


def _norm2_jk_body(x_ref, h1_ref, a0_ref, a1_ref, s0_ref, s1_ref, b2_ref,
                   wjk_ref, bjk_ref, out_ref, *, R):
    acc = a0_ref[...] + a1_ref[...]
    s = s0_ref[...] + s1_ref[...]
    h2 = _norm_block(acc, s, b2_ref[...], 2, 4, R)
    w = wjk_ref[...]
    o = jnp.dot(x_ref[...], w[0:128], preferred_element_type=F32)
    o = o + jnp.dot(h1_ref[...], w[128:640], preferred_element_type=F32)
    o = o + jnp.dot(h2, w[640:896], preferred_element_type=F32)
    out_ref[...] = o + bjk_ref[...]


def _norm2_jk(x, h1, acc2a, acc2b, s2a, s2b, bias2, Wjk, bjk, N, R):
    grid = (N // R,)
    return pl.pallas_call(
        functools.partial(_norm2_jk_body, R=R),
        grid=grid,
        in_specs=[
            pl.BlockSpec((R, 128), lambda r: (r, 0)),
            pl.BlockSpec((R, 512), lambda r: (r, 0)),
            pl.BlockSpec((2, R, 128), lambda r: (0, r, 0)),
            pl.BlockSpec((2, R, 128), lambda r: (0, r, 0)),
            pl.BlockSpec((2, R, 16), lambda r: (0, r, 0)),
            pl.BlockSpec((2, R, 16), lambda r: (0, r, 0)),
            pl.BlockSpec((2, 128), lambda r: (0, 0)),
            pl.BlockSpec((896, 128), lambda r: (0, 0)),
            pl.BlockSpec((1, 128), lambda r: (0, 0)),
        ],
        out_specs=pl.BlockSpec((R, 128), lambda r: (r, 0)),
        out_shape=jax.ShapeDtypeStruct((N, 128), F32),
    )(x, h1, acc2a, acc2b, s2a, s2b, bias2.reshape(2, 128), Wjk,
      bjk.reshape(1, 128))


def kernel(x, edge_index, Wl1, bl1, Wr1, br1, att1, bias1,
           Wl2, bl2, Wr2, br2, att2, bias2, Wjk, bjk):
    N = x.shape[0]
    E = edge_index.shape[1]
    E_real = E + N
    q = NTEC * K * 2
    Epad = ((E_real + q - 1) // q) * q
    NP = ((N + TPS * 8 - 1) // (TPS * 8)) * (TPS * 8)
    R = 1000

    loop = jnp.arange(N, dtype=edge_index.dtype)
    pad = jnp.zeros((Epad - E_real,), dtype=edge_index.dtype)
    src = jnp.concatenate([edge_index[0], loop, pad])
    dst = jnp.concatenate([edge_index[1], loop, pad])

    zf = jnp.zeros((NP, 128), F32)
    z16 = jnp.zeros((NP, 16), F32)

    xl1, xr1 = _mm_in(x, Wl1, Wr1, bl1, br1, 4, R)
    att1g = att1.reshape(4, 128)
    ek1 = _make_sc_edge(4, 2, N, NP, Epad, E_real)
    acc1, s1 = ek1(xl1.reshape(4 * N, 128), xr1.reshape(4 * N, 128),
                   src, dst, att1g, zf, z16)
    acc1 = acc1.reshape(NSC, 4, NP, 128)[:, :, :N]
    s1 = s1.reshape(NSC, 4, NP, 16)[:, :, :N]

    h1, xl2, xr2 = _norm1_mm2((acc1[0], acc1[1]), (s1[0], s1[1]), bias1,
                              Wl2, Wr2, bl2, br2, N, R)

    att2g = att2.reshape(2, 128)
    ek2 = _make_sc_edge(2, 4, N, NP, Epad, E_real)
    acc2, s2 = ek2(xl2.reshape(2 * N, 128), xr2.reshape(2 * N, 128),
                   src, dst, att2g, zf, z16)
    acc2 = acc2.reshape(NSC, 2, NP, 128)[:, :, :N]
    s2 = s2.reshape(NSC, 2, NP, 16)[:, :, :N]

    return _norm2_jk(x, h1, acc2[0], acc2[1], s2[0], s2[1], bias2,
                     Wjk, bjk, N, R)


# speedup vs baseline: 1.2061x; 1.0131x over previous
import functools
import jax
import jax.numpy as jnp
from jax import lax
from jax.experimental import pallas as pl
from jax.experimental.pallas import tpu as pltpu
from jax.experimental.pallas import tpu_sc as plsc

F32 = jnp.float32
K = 64
NTEC = 32
NSC = 2
TPS = 16


def _make_sc_edge(nG, G, N, NP, Epad, E_real):
    CV = 128 // (16 * G)
    EP = Epad // NTEC
    nch = EP // K
    RPT = NP // TPS

    mesh = plsc.VectorSubcoreMesh(core_axis_name="c", subcore_axis_name="s")

    @functools.partial(
        pl.kernel,
        out_type=(
            jax.ShapeDtypeStruct((NSC * nG * NP, 128), F32),
            jax.ShapeDtypeStruct((NSC * nG * NP, 16), F32),
        ),
        mesh=mesh,
        compiler_params=pltpu.CompilerParams(
            needs_layout_passes=False, use_tc_tiling_on_sc=False),
        scratch_types=[
            pltpu.VMEM_SHARED((NP, 128), F32),
            pltpu.VMEM_SHARED((NP, 16), F32),
            pltpu.VMEM((K,), jnp.int32),
            pltpu.VMEM((K,), jnp.int32),
            pltpu.VMEM((K,), jnp.int32),
            pltpu.VMEM((K,), jnp.int32),
            pltpu.VMEM((K,), jnp.int32),
            pltpu.VMEM((K,), jnp.int32),
            pltpu.VMEM((K,), jnp.int32),
            pltpu.VMEM((K,), jnp.int32),
            pltpu.VMEM((K, 128), F32),
            pltpu.VMEM((K, 128), F32),
            pltpu.VMEM((K, 128), F32),
            pltpu.VMEM((K, 128), F32),
            pltpu.VMEM((K, 16), F32),
            pltpu.VMEM((K, 16), F32),
            pltpu.VMEM((16, 16), F32),
            pltpu.VMEM((128,), F32),
            pltpu.SemaphoreType.DMA,
            pltpu.SemaphoreType.DMA,
            pltpu.SemaphoreType.DMA,
            pltpu.SemaphoreType.DMA,
            pltpu.SemaphoreType.DMA,
            pltpu.SemaphoreType.DMA,
            pltpu.SemaphoreType.DMA,
        ],
    )
    def edge_kernel(xl_hbm, xr_hbm, src_hbm, dst_hbm, att_hbm, zf_hbm, z16_hbm,
                    acc_out, s_out,
                    acc_sp, s_sp, srcv0, srcv1, dstv0, dstv1, idxl0, idxl1,
                    idxr0, idxr1, xlv0, xlv1, xrv0, xrv1, wv0, wv1, tbuf,
                    attc, gsl0, gsl1, gsr0, gsr1, isem0, isem1, ssem):
        c = lax.axis_index("c")
        t = lax.axis_index("s")
        wid = c * TPS + t
        ebase = wid * EP
        rbase = t * RPT
        lane = lax.iota(jnp.int32, 16)
        Ch = 128 // G
        bufs0 = (srcv0, dstv0, idxl0, idxr0, xlv0, xrv0, wv0, gsl0, gsr0,
                 isem0)
        bufs1 = (srcv1, dstv1, idxl1, idxr1, xlv1, xrv1, wv1, gsl1, gsr1,
                 isem1)

        def wz_body(i, _):
            wv0[i, pl.ds(0, 16)] = jnp.zeros((16,), F32)
            wv1[i, pl.ds(0, 16)] = jnp.zeros((16,), F32)
            return 0

        lax.fori_loop(0, K, wz_body, 0)

        def idx_load_async(k, bufs):
            srcv, dstv, isem = bufs[0], bufs[1], bufs[9]
            base = ebase + k * K
            pltpu.async_copy(src_hbm.at[pl.ds(base, K)], srcv, isem)
            pltpu.async_copy(dst_hbm.at[pl.ds(base, K)], dstv, isem)

        def build_and_gather(k, gN, bufs):
            srcv, dstv, idxl, idxr, xlv, xrv, _, gsl, gsr, isem = bufs
            base = ebase + k * K
            pltpu.make_async_copy(
                src_hbm.at[pl.ds(base, K)], srcv, isem).wait()
            pltpu.make_async_copy(
                dst_hbm.at[pl.ds(base, K)], dstv, isem).wait()
            for j in range(K // 16):
                sl = pl.ds(j * 16, 16)
                idxl[sl] = srcv[sl] + gN
                idxr[sl] = dstv[sl] + gN
            pltpu.async_copy(xl_hbm.at[idxl], xlv, gsl)
            pltpu.async_copy(xr_hbm.at[idxr], xrv, gsr)

        def group_body(g, _):
            gN = g * N
            pltpu.sync_copy(att_hbm.at[g], attc)
            attvecs = [attc[pl.ds(j * 16, 16)] for j in range(8)]
            pltpu.sync_copy(zf_hbm.at[pl.ds(rbase, RPT)],
                            acc_sp.at[pl.ds(rbase, RPT)])
            pltpu.sync_copy(z16_hbm.at[pl.ds(rbase, RPT)],
                            s_sp.at[pl.ds(rbase, RPT)])
            plsc.subcore_barrier()

            def chunk_slot(k, bufs, nbufs):
                srcv, dstv, idxl, idxr, xlv, xrv, wv, gsl, gsr, isem = bufs
                base = ebase + k * K
                pltpu.make_async_copy(xl_hbm.at[idxl], xlv, gsl).wait()
                pltpu.make_async_copy(xr_hbm.at[idxr], xrv, gsr).wait()

                @pl.when(k + 1 < nch)
                def _():
                    build_and_gather(k + 1, gN, nbufs)

                def blk_body(eb, _):
                    rows = eb * 16 + lane
                    mvec = jnp.where(base + rows < E_real, 1.0, 0.0
                                     ).astype(F32)
                    for h in range(G):
                        for j in range(16):
                            e = eb * 16 + j
                            psum = None
                            for q in range(CV):
                                sl = pl.ds((h * CV + q) * 16, 16)
                                z = xlv[e, sl] + xrv[e, sl]
                                lk = jnp.maximum(z, z * 0.2)
                                term = lk * attvecs[h * CV + q]
                                psum = term if psum is None else psum + term
                            tbuf[j, pl.ds(0, 16)] = psum
                        cols = [
                            plsc.load_gather(
                                tbuf,
                                [lane, jnp.full((16,), j, jnp.int32)])
                            for j in range(16)
                        ]
                        while len(cols) > 1:
                            cols = [cols[i] + cols[i + 1]
                                    for i in range(0, len(cols), 2)]
                        w = jnp.exp(cols[0]) * mvec
                        plsc.store_scatter(
                            wv, [rows, jnp.full((16,), h, jnp.int32)], w)
                        for j in range(16):
                            e = eb * 16 + j
                            ws = w[j]
                            for q in range(CV):
                                sl = pl.ds((h * CV + q) * 16, 16)
                                xrv[e, sl] = xlv[e, sl] * ws
                    return 0

                lax.fori_loop(0, K // 16, blk_body, 0)

                cacc = pltpu.async_copy(xrv, acc_sp.at[dstv], ssem, add=True)
                pltpu.sync_copy(wv, s_sp.at[dstv], add=True)
                cacc.wait()

                @pl.when(k + 2 < nch)
                def _():
                    idx_load_async(k + 2, bufs)

            idx_load_async(0, bufs0)
            idx_load_async(1, bufs1)
            build_and_gather(0, gN, bufs0)

            def pair_body(i, _):
                chunk_slot(2 * i, bufs0, bufs1)
                chunk_slot(2 * i + 1, bufs1, bufs0)
                return 0

            lax.fori_loop(0, nch // 2, pair_body, 0)
            plsc.subcore_barrier()
            obase = (c * nG + g) * NP + rbase
            pltpu.sync_copy(acc_sp.at[pl.ds(rbase, RPT)],
                            acc_out.at[pl.ds(obase, RPT)])
            pltpu.sync_copy(s_sp.at[pl.ds(rbase, RPT)],
                            s_out.at[pl.ds(obase, RPT)])
            plsc.subcore_barrier()
            return 0

        lax.fori_loop(0, nG, group_body, 0)

    return edge_kernel


def _mm_in_body(x_ref, wl_ref, wr_ref, bl_ref, br_ref, xl_ref, xr_ref):
    xb = x_ref[...]
    xl_ref[0] = jnp.dot(xb, wl_ref[...], preferred_element_type=F32) + bl_ref[0]
    xr_ref[0] = jnp.dot(xb, wr_ref[...], preferred_element_type=F32) + br_ref[0]


def _mm_in(x, Wl, Wr, bl, br, nG, R):
    N = x.shape[0]
    HIDD = x.shape[1]
    grid = (nG, N // R)
    out_shape = [
        jax.ShapeDtypeStruct((nG, N, 128), F32),
        jax.ShapeDtypeStruct((nG, N, 128), F32),
    ]
    return pl.pallas_call(
        _mm_in_body,
        grid=grid,
        in_specs=[
            pl.BlockSpec((R, HIDD), lambda g, r: (r, 0)),
            pl.BlockSpec((HIDD, 128), lambda g, r: (0, g)),
            pl.BlockSpec((HIDD, 128), lambda g, r: (0, g)),
            pl.BlockSpec((1, 1, 128), lambda g, r: (g, 0, 0)),
            pl.BlockSpec((1, 1, 128), lambda g, r: (g, 0, 0)),
        ],
        out_specs=[
            pl.BlockSpec((1, R, 128), lambda g, r: (g, r, 0)),
            pl.BlockSpec((1, R, 128), lambda g, r: (g, r, 0)),
        ],
        out_shape=out_shape,
    )(x, Wl, Wr, bl.reshape(nG, 1, 128), br.reshape(nG, 1, 128))


def _elu(v):
    return jnp.where(v > 0, v, jnp.exp(v) - 1.0)


def _norm_block(acc, s, bias_ref, nG, G, R):
    C = 128 // G
    col = lax.broadcasted_iota(jnp.int32, (R, 128), 1)
    outs = []
    for g in range(nG):
        den = jnp.zeros((R, 128), F32)
        for h in range(G):
            sh = s[g, :, h:h + 1]
            m = jnp.logical_and(col >= h * C, col < (h + 1) * C)
            den = jnp.where(m, sh, den)
        hg = acc[g] / den + bias_ref[g]
        outs.append(_elu(hg))
    return jnp.concatenate(outs, axis=1)


def _norm1_mm2_body(a0_ref, a1_ref, s0_ref, s1_ref, b1_ref, w2_ref, b2_ref,
                    h1_ref, xl2_ref, xr2_ref, *, R):
    acc = a0_ref[...] + a1_ref[...]
    s = s0_ref[...] + s1_ref[...]
    h1 = _norm_block(acc, s, b1_ref[...], 4, 2, R)
    h1_ref[...] = h1
    xw = jnp.dot(h1, w2_ref[...], preferred_element_type=F32)
    b2 = b2_ref[...]
    for q in range(2):
        xl2_ref[q] = xw[:, q * 128:(q + 1) * 128] + b2[q]
        xr2_ref[q] = xw[:, 256 + q * 128:256 + (q + 1) * 128] + b2[2 + q]


def _norm1_mm2(acc1, s1, bias1, Wl2, Wr2, bl2, br2, N, R):
    W2 = jnp.concatenate([Wl2, Wr2], axis=1)
    b2 = jnp.concatenate([bl2, br2]).reshape(4, 128)
    grid = (N // R,)
    out_shape = [
        jax.ShapeDtypeStruct((N, 512), F32),
        jax.ShapeDtypeStruct((2, N, 128), F32),
        jax.ShapeDtypeStruct((2, N, 128), F32),
    ]
    return pl.pallas_call(
        functools.partial(_norm1_mm2_body, R=R),
        grid=grid,
        in_specs=[
            pl.BlockSpec((4, R, 128), lambda r: (0, r, 0)),
            pl.BlockSpec((4, R, 128), lambda r: (0, r, 0)),
            pl.BlockSpec((4, R, 16), lambda r: (0, r, 0)),
            pl.BlockSpec((4, R, 16), lambda r: (0, r, 0)),
            pl.BlockSpec((4, 128), lambda r: (0, 0)),
            pl.BlockSpec((512, 512), lambda r: (0, 0)),
            pl.BlockSpec((4, 128), lambda r: (0, 0)),
        ],
        out_specs=[
            pl.BlockSpec((R, 512), lambda r: (r, 0)),
            pl.BlockSpec((2, R, 128), lambda r: (0, r, 0)),
            pl.BlockSpec((2, R, 128), lambda r: (0, r, 0)),
        ],
        out_shape=out_shape,
    )(acc1[0], acc1[1], s1[0], s1[1], bias1.reshape(4, 128), W2, b2)


def _norm2_jk_body(x_ref, h1_ref, a0_ref, a1_ref, s0_ref, s1_ref, b2_ref,
                   wjk_ref, bjk_ref, out_ref, *, R):
    acc = a0_ref[...] + a1_ref[...]
    s = s0_ref[...] + s1_ref[...]
    h2 = _norm_block(acc, s, b2_ref[...], 2, 4, R)
    w = wjk_ref[...]
    o = jnp.dot(x_ref[...], w[0:128], preferred_element_type=F32)
    o = o + jnp.dot(h1_ref[...], w[128:640], preferred_element_type=F32)
    o = o + jnp.dot(h2, w[640:896], preferred_element_type=F32)
    out_ref[...] = o + bjk_ref[...]


def _norm2_jk(x, h1, acc2a, acc2b, s2a, s2b, bias2, Wjk, bjk, N, R):
    grid = (N // R,)
    return pl.pallas_call(
        functools.partial(_norm2_jk_body, R=R),
        grid=grid,
        in_specs=[
            pl.BlockSpec((R, 128), lambda r: (r, 0)),
            pl.BlockSpec((R, 512), lambda r: (r, 0)),
            pl.BlockSpec((2, R, 128), lambda r: (0, r, 0)),
            pl.BlockSpec((2, R, 128), lambda r: (0, r, 0)),
            pl.BlockSpec((2, R, 16), lambda r: (0, r, 0)),
            pl.BlockSpec((2, R, 16), lambda r: (0, r, 0)),
            pl.BlockSpec((2, 128), lambda r: (0, 0)),
            pl.BlockSpec((896, 128), lambda r: (0, 0)),
            pl.BlockSpec((1, 128), lambda r: (0, 0)),
        ],
        out_specs=pl.BlockSpec((R, 128), lambda r: (r, 0)),
        out_shape=jax.ShapeDtypeStruct((N, 128), F32),
    )(x, h1, acc2a, acc2b, s2a, s2b, bias2.reshape(2, 128), Wjk,
      bjk.reshape(1, 128))


def kernel(x, edge_index, Wl1, bl1, Wr1, br1, att1, bias1,
           Wl2, bl2, Wr2, br2, att2, bias2, Wjk, bjk):
    N = x.shape[0]
    E = edge_index.shape[1]
    E_real = E + N
    q = NTEC * K * 2
    Epad = ((E_real + q - 1) // q) * q
    NP = ((N + TPS * 8 - 1) // (TPS * 8)) * (TPS * 8)
    R = 1000

    loop = jnp.arange(N, dtype=edge_index.dtype)
    pad = jnp.zeros((Epad - E_real,), dtype=edge_index.dtype)
    src = jnp.concatenate([edge_index[0], loop, pad])
    dst = jnp.concatenate([edge_index[1], loop, pad])

    zf = jnp.zeros((NP, 128), F32)
    z16 = jnp.zeros((NP, 16), F32)

    xl1, xr1 = _mm_in(x, Wl1, Wr1, bl1, br1, 4, R)
    att1g = att1.reshape(4, 128)
    ek1 = _make_sc_edge(4, 2, N, NP, Epad, E_real)
    acc1, s1 = ek1(xl1.reshape(4 * N, 128), xr1.reshape(4 * N, 128),
                   src, dst, att1g, zf, z16)
    acc1 = acc1.reshape(NSC, 4, NP, 128)[:, :, :N]
    s1 = s1.reshape(NSC, 4, NP, 16)[:, :, :N]

    h1, xl2, xr2 = _norm1_mm2((acc1[0], acc1[1]), (s1[0], s1[1]), bias1,
                              Wl2, Wr2, bl2, br2, N, R)

    att2g = att2.reshape(2, 128)
    ek2 = _make_sc_edge(2, 4, N, NP, Epad, E_real)
    acc2, s2 = ek2(xl2.reshape(2 * N, 128), xr2.reshape(2 * N, 128),
                   src, dst, att2g, zf, z16)
    acc2 = acc2.reshape(NSC, 2, NP, 128)[:, :, :N]
    s2 = s2.reshape(NSC, 2, NP, 16)[:, :, :N]

    return _norm2_jk(x, h1, acc2[0], acc2[1], s2[0], s2[1], bias2,
                     Wjk, bjk, N, R)
